# Initial kernel scaffold; baseline (speedup 1.0000x reference)
import jax
import jax.numpy as jnp
from jax.experimental import pallas as pl


def kernel(user_tokens, context_tokens, candidate_tokens, candidate_post_tokens, candidate_author_tokens, dense_features, table_user_tokens, table_context_tokens, table_candidate_tokens, table_candidate_post_tokens, table_candidate_author_tokens, W1, b1, W2, b2):
    raise NotImplementedError("write your pallas kernel here")


# speedup vs baseline: 1.3808x; 1.3808x over previous
import functools

import jax
import jax.numpy as jnp
from jax import lax
from jax.experimental import pallas as pl
from jax.experimental.pallas import tpu as pltpu
from jax.experimental.pallas import tpu_sc as plsc

B = 4096
H = 20
E = 64
NTAB = 5
DENSE = 256
HIDDEN = 512

NC, NS, L = 2, 16, 16
NW = NC * NS
BAGS_W = B // NW
CHUNK = 64
NCHUNK = BAGS_W // CHUNK
ROWS_CHUNK = CHUNK * H


def _sc_pool(idx0, idx1, idx2, idx3, idx4, t0, t1, t2, t3, t4):
    mesh = plsc.VectorSubcoreMesh(core_axis_name="c", subcore_axis_name="s")

    @functools.partial(
        pl.kernel,
        out_type=jax.ShapeDtypeStruct((NTAB, B, E), jnp.float32),
        mesh=mesh,
        scratch_types=[
            pltpu.VMEM((BAGS_W * H,), jnp.int32),
            pltpu.VMEM((ROWS_CHUNK, E), jnp.float32),
            pltpu.VMEM((BAGS_W, E), jnp.float32),
            pltpu.SemaphoreType.DMA,
        ],
        compiler_params=pltpu.CompilerParams(use_tc_tiling_on_sc=False),
    )
    def k(i0, i1, i2, i3, i4, tb0, tb1, tb2, tb3, tb4, out_hbm,
          idx_v, rows_v, pool_v, sem):
        wid = lax.axis_index("s") * NC + lax.axis_index("c")
        base_bag = wid * BAGS_W
        base_idx = base_bag * H
        for t, (ihbm, thbm) in enumerate(
                zip((i0, i1, i2, i3, i4), (tb0, tb1, tb2, tb3, tb4))):
            pltpu.sync_copy(ihbm.at[pl.ds(base_idx, BAGS_W * H)], idx_v)
            for c in range(NCHUNK):
                pltpu.async_copy(
                    thbm.at[idx_v.at[pl.ds(c * ROWS_CHUNK, ROWS_CHUNK)]],
                    rows_v, sem).wait()

                def body(bag, carry, _c=c):
                    r0 = bag * H
                    for j in range(E // L):
                        acc = rows_v[r0, pl.ds(j * L, L)]
                        for q in range(1, H):
                            acc = acc + rows_v[r0 + q, pl.ds(j * L, L)]
                        pool_v[_c * CHUNK + bag, pl.ds(j * L, L)] = acc
                    return carry

                lax.fori_loop(0, CHUNK, body, 0)
            pltpu.sync_copy(pool_v, out_hbm.at[t, pl.ds(base_bag, BAGS_W)])

    return k(idx0, idx1, idx2, idx3, idx4, t0, t1, t2, t3, t4)


def _mlp(pooled, dense, W1, b1r, W2r, b2r):
    BLK = 512
    FUSED = NTAB * E + DENSE

    def body(p_ref, d_ref, w1_ref, b1_ref, w2_ref, b2_ref, o_ref):
        h = jnp.dot(d_ref[...], w1_ref[NTAB * E:, :],
                    preferred_element_type=jnp.float32)
        for t in range(NTAB):
            h = h + jnp.dot(p_ref[t], w1_ref[t * E:(t + 1) * E, :],
                            preferred_element_type=jnp.float32)
        h = h + b1_ref[...]
        h = h * jax.nn.sigmoid(h)
        o_ref[...] = jnp.sum(h * w2_ref[...], axis=1, keepdims=True) + b2_ref[...]

    return pl.pallas_call(
        body,
        grid=(B // BLK,),
        in_specs=[
            pl.BlockSpec((NTAB, BLK, E), lambda i: (0, i, 0)),
            pl.BlockSpec((BLK, DENSE), lambda i: (i, 0)),
            pl.BlockSpec((FUSED, HIDDEN), lambda i: (0, 0)),
            pl.BlockSpec((1, HIDDEN), lambda i: (0, 0)),
            pl.BlockSpec((1, HIDDEN), lambda i: (0, 0)),
            pl.BlockSpec((1, 1), lambda i: (0, 0)),
        ],
        out_specs=pl.BlockSpec((BLK, 1), lambda i: (i, 0)),
        out_shape=jax.ShapeDtypeStruct((B, 1), jnp.float32),
    )(pooled, dense, W1, b1r, W2r, b2r)


def kernel(user_tokens, context_tokens, candidate_tokens,
           candidate_post_tokens, candidate_author_tokens, dense_features,
           table_user_tokens, table_context_tokens, table_candidate_tokens,
           table_candidate_post_tokens, table_candidate_author_tokens,
           W1, b1, W2, b2):
    idx = [jnp.reshape(t, (B * H,)).astype(jnp.int32)
           for t in (user_tokens, context_tokens, candidate_tokens,
                     candidate_post_tokens, candidate_author_tokens)]
    pooled = _sc_pool(*idx, table_user_tokens, table_context_tokens,
                      table_candidate_tokens, table_candidate_post_tokens,
                      table_candidate_author_tokens)
    out = _mlp(pooled, dense_features, W1,
               jnp.reshape(b1, (1, HIDDEN)),
               jnp.reshape(W2, (1, HIDDEN)),
               jnp.reshape(b2, (1, 1)))
    return jnp.squeeze(out, axis=-1)


# speedup vs baseline: 1.4936x; 1.0817x over previous
import functools

import jax
import jax.numpy as jnp
from jax import lax
from jax.experimental import pallas as pl
from jax.experimental.pallas import tpu as pltpu
from jax.experimental.pallas import tpu_sc as plsc

B = 4096
H = 20
E = 64
NTAB = 5
DENSE = 256
HIDDEN = 512

NC, NS, L = 2, 16, 16
NW = NC * NS
BAGS_W = B // NW
CHUNK = 32
NCHUNK = BAGS_W // CHUNK
ROWS_CHUNK = CHUNK * H


def _sc_pool(idx0, idx1, idx2, idx3, idx4, t0, t1, t2, t3, t4):
    mesh = plsc.VectorSubcoreMesh(core_axis_name="c", subcore_axis_name="s")

    @functools.partial(
        pl.kernel,
        out_type=jax.ShapeDtypeStruct((NTAB, B, E), jnp.float32),
        mesh=mesh,
        scratch_types=[
            pltpu.VMEM((NTAB, BAGS_W * H), jnp.int32),
            pltpu.VMEM((2, ROWS_CHUNK, E), jnp.float32),
            pltpu.VMEM((BAGS_W, E), jnp.float32),
            pltpu.SemaphoreType.DMA,
            pltpu.SemaphoreType.DMA,
            pltpu.SemaphoreType.DMA,
        ],
        compiler_params=pltpu.CompilerParams(use_tc_tiling_on_sc=False),
    )
    def k(i0, i1, i2, i3, i4, tb0, tb1, tb2, tb3, tb4, out_hbm,
          idx_v, rows_v, pool_v, sem0, sem1, osem):
        wid = lax.axis_index("s") * NC + lax.axis_index("c")
        base_bag = wid * BAGS_W
        base_idx = base_bag * H
        ihbms = (i0, i1, i2, i3, i4)
        thbms = (tb0, tb1, tb2, tb3, tb4)
        sems = (sem0, sem1)
        for t in range(NTAB):
            pltpu.sync_copy(ihbms[t].at[pl.ds(base_idx, BAGS_W * H)],
                            idx_v.at[t])
        steps = [(t, c) for t in range(NTAB) for c in range(NCHUNK)]

        def start(step, slot):
            t, c = steps[step]
            return pltpu.async_copy(
                thbms[t].at[idx_v.at[t, pl.ds(c * ROWS_CHUNK, ROWS_CHUNK)]],
                rows_v.at[slot], sems[slot])

        handles = {0: start(0, 0)}
        for s, (t, c) in enumerate(steps):
            slot = s % 2
            if s + 1 < len(steps):
                handles[s + 1] = start(s + 1, 1 - slot)
            handles.pop(s).wait()

            def body(bag, carry, _c=c, _slot=slot):
                r0 = bag * H
                for j in range(E // L):
                    acc = rows_v[_slot, r0, pl.ds(j * L, L)]
                    for q in range(1, H):
                        acc = acc + rows_v[_slot, r0 + q, pl.ds(j * L, L)]
                    pool_v[_c * CHUNK + bag, pl.ds(j * L, L)] = acc
                return carry

            lax.fori_loop(0, CHUNK, body, 0)
            if c == NCHUNK - 1:
                pltpu.async_copy(
                    pool_v, out_hbm.at[t, pl.ds(base_bag, BAGS_W)],
                    osem).wait()

    return k(idx0, idx1, idx2, idx3, idx4, t0, t1, t2, t3, t4)


def _mlp(pooled, dense, W1, b1r, W2r, b2r):
    BLK = 512
    FUSED = NTAB * E + DENSE

    def body(p_ref, d_ref, w1_ref, b1_ref, w2_ref, b2_ref, o_ref):
        h = jnp.dot(d_ref[...], w1_ref[NTAB * E:, :],
                    preferred_element_type=jnp.float32)
        for t in range(NTAB):
            h = h + jnp.dot(p_ref[t], w1_ref[t * E:(t + 1) * E, :],
                            preferred_element_type=jnp.float32)
        h = h + b1_ref[...]
        h = h * jax.nn.sigmoid(h)
        o_ref[...] = jnp.sum(h * w2_ref[...], axis=1, keepdims=True) + b2_ref[...]

    return pl.pallas_call(
        body,
        grid=(B // BLK,),
        in_specs=[
            pl.BlockSpec((NTAB, BLK, E), lambda i: (0, i, 0)),
            pl.BlockSpec((BLK, DENSE), lambda i: (i, 0)),
            pl.BlockSpec((FUSED, HIDDEN), lambda i: (0, 0)),
            pl.BlockSpec((1, HIDDEN), lambda i: (0, 0)),
            pl.BlockSpec((1, HIDDEN), lambda i: (0, 0)),
            pl.BlockSpec((1, 1), lambda i: (0, 0)),
        ],
        out_specs=pl.BlockSpec((BLK, 1), lambda i: (i, 0)),
        out_shape=jax.ShapeDtypeStruct((B, 1), jnp.float32),
    )(pooled, dense, W1, b1r, W2r, b2r)


def kernel(user_tokens, context_tokens, candidate_tokens,
           candidate_post_tokens, candidate_author_tokens, dense_features,
           table_user_tokens, table_context_tokens, table_candidate_tokens,
           table_candidate_post_tokens, table_candidate_author_tokens,
           W1, b1, W2, b2):
    idx = [jnp.reshape(t, (B * H,)).astype(jnp.int32)
           for t in (user_tokens, context_tokens, candidate_tokens,
                     candidate_post_tokens, candidate_author_tokens)]
    pooled = _sc_pool(*idx, table_user_tokens, table_context_tokens,
                      table_candidate_tokens, table_candidate_post_tokens,
                      table_candidate_author_tokens)
    out = _mlp(pooled, dense_features, W1,
               jnp.reshape(b1, (1, HIDDEN)),
               jnp.reshape(W2, (1, HIDDEN)),
               jnp.reshape(b2, (1, 1)))
    return jnp.squeeze(out, axis=-1)


# speedup vs baseline: 1.8700x; 1.2520x over previous
import functools

import jax
import jax.numpy as jnp
from jax import lax
from jax.experimental import pallas as pl
from jax.experimental.pallas import tpu as pltpu
from jax.experimental.pallas import tpu_sc as plsc

B = 4096
H = 20
E = 64
V = 100000
NTAB = 5
DENSE = 256
HIDDEN = 512

VP = 102400
PHALF = VP // 2
TBLK = 2048

NC, NS, L = 2, 16, 16
NW = NC * NS
BAGS_W = B // NW
CHUNK = 32
NCHUNK = BAGS_W // CHUNK
ROWS_CHUNK = CHUNK * H


def _sc_pool(idx0, idx1, idx2, idx3, idx4, t0, t1, t2, t3, t4):
    mesh = plsc.VectorSubcoreMesh(core_axis_name="c", subcore_axis_name="s")

    @functools.partial(
        pl.kernel,
        out_type=jax.ShapeDtypeStruct((NTAB, B, E), jnp.float32),
        mesh=mesh,
        scratch_types=[
            pltpu.VMEM((NTAB, BAGS_W * H), jnp.int32),
            pltpu.VMEM((2, ROWS_CHUNK, E), jnp.float32),
            pltpu.VMEM((BAGS_W, E), jnp.float32),
            pltpu.SemaphoreType.DMA,
            pltpu.SemaphoreType.DMA,
            pltpu.SemaphoreType.DMA,
        ],
        compiler_params=pltpu.CompilerParams(use_tc_tiling_on_sc=False),
    )
    def k(i0, i1, i2, i3, i4, tb0, tb1, tb2, tb3, tb4, out_hbm,
          idx_v, rows_v, pool_v, sem0, sem1, osem):
        wid = lax.axis_index("s") * NC + lax.axis_index("c")
        base_bag = wid * BAGS_W
        base_idx = base_bag * H
        ihbms = (i0, i1, i2, i3, i4)
        thbms = (tb0, tb1, tb2, tb3, tb4)
        sems = (sem0, sem1)
        for t in range(NTAB):
            pltpu.sync_copy(ihbms[t].at[pl.ds(base_idx, BAGS_W * H)],
                            idx_v.at[t])
        steps = [(t, c) for t in range(NTAB) for c in range(NCHUNK)]

        def start(step, slot):
            t, c = steps[step]
            return pltpu.async_copy(
                thbms[t].at[idx_v.at[t, pl.ds(c * ROWS_CHUNK, ROWS_CHUNK)]],
                rows_v.at[slot], sems[slot])

        handles = {0: start(0, 0)}
        for s, (t, c) in enumerate(steps):
            slot = s % 2
            if s + 1 < len(steps):
                handles[s + 1] = start(s + 1, 1 - slot)
            handles.pop(s).wait()

            def body(bag, carry, _c=c, _slot=slot):
                r0 = bag * H
                for j in range(E // L):
                    acc = rows_v[_slot, r0, pl.ds(j * L, L)]
                    for q in range(1, H):
                        acc = acc + rows_v[_slot, r0 + q, pl.ds(j * L, L)]
                    pool_v[_c * CHUNK + bag, pl.ds(j * L, L)] = acc
                return carry

            lax.fori_loop(0, CHUNK, body, 0)
            if c == NCHUNK - 1:
                pltpu.async_copy(
                    pool_v, out_hbm.at[t, pl.ds(base_bag, BAGS_W)],
                    osem).wait()

    return k(idx0, idx1, idx2, idx3, idx4, t0, t1, t2, t3, t4)


def _tc_repack(tT, eye):
    def body(x1_ref, x2_ref, i_ref, o_ref):
        dn = (((0,), (0,)), ((), ()))
        o_ref[:, 0:E] = jax.lax.dot_general(
            x1_ref[...], i_ref[...], dn, preferred_element_type=jnp.float32)
        o_ref[:, E:2 * E] = jax.lax.dot_general(
            x2_ref[...], i_ref[...], dn, preferred_element_type=jnp.float32)

    nblk = PHALF // TBLK
    last = -(-V // TBLK) - 1
    return pl.pallas_call(
        body,
        grid=(nblk,),
        in_specs=[
            pl.BlockSpec((E, TBLK), lambda i: (0, i)),
            pl.BlockSpec((E, TBLK), lambda i: (0, jnp.minimum(i + nblk, last))),
            pl.BlockSpec((E, E), lambda i: (0, 0)),
        ],
        out_specs=pl.BlockSpec((TBLK, 2 * E), lambda i: (i, 0)),
        out_shape=jax.ShapeDtypeStruct((PHALF, 2 * E), jnp.float32),
    )(tT, tT, eye)


def _mlp(pooled, dense, W1, b1r, W2r, b2r):
    BLK = 512
    FUSED = NTAB * E + DENSE

    def body(p_ref, d_ref, w1_ref, b1_ref, w2_ref, b2_ref, o_ref):
        h = jnp.dot(d_ref[...], w1_ref[NTAB * E:, :],
                    preferred_element_type=jnp.float32)
        for t in range(NTAB):
            h = h + jnp.dot(p_ref[t], w1_ref[t * E:(t + 1) * E, :],
                            preferred_element_type=jnp.float32)
        h = h + b1_ref[...]
        h = h * jax.nn.sigmoid(h)
        o_ref[...] = jnp.sum(h * w2_ref[...], axis=1, keepdims=True) + b2_ref[...]

    return pl.pallas_call(
        body,
        grid=(B // BLK,),
        in_specs=[
            pl.BlockSpec((NTAB, BLK, E), lambda i: (0, i, 0)),
            pl.BlockSpec((BLK, DENSE), lambda i: (i, 0)),
            pl.BlockSpec((FUSED, HIDDEN), lambda i: (0, 0)),
            pl.BlockSpec((1, HIDDEN), lambda i: (0, 0)),
            pl.BlockSpec((1, HIDDEN), lambda i: (0, 0)),
            pl.BlockSpec((1, 1), lambda i: (0, 0)),
        ],
        out_specs=pl.BlockSpec((BLK, 1), lambda i: (i, 0)),
        out_shape=jax.ShapeDtypeStruct((B, 1), jnp.float32),
    )(pooled, dense, W1, b1r, W2r, b2r)


def kernel(user_tokens, context_tokens, candidate_tokens,
           candidate_post_tokens, candidate_author_tokens, dense_features,
           table_user_tokens, table_context_tokens, table_candidate_tokens,
           table_candidate_post_tokens, table_candidate_author_tokens,
           W1, b1, W2, b2):
    idx = []
    for t in (user_tokens, context_tokens, candidate_tokens,
              candidate_post_tokens, candidate_author_tokens):
        t = t.astype(jnp.int32)
        t = jnp.where(t < PHALF, t * 2, t * 2 - (VP - 1))
        idx.append(jnp.reshape(t, (B * H,)))
    eye = jnp.eye(E, dtype=jnp.float32)
    tabs = [jnp.reshape(_tc_repack(tbl.T, eye), (VP, E))
            for tbl in (table_user_tokens, table_context_tokens,
                        table_candidate_tokens, table_candidate_post_tokens,
                        table_candidate_author_tokens)]
    pooled = _sc_pool(*idx, *tabs)
    out = _mlp(pooled, dense_features, W1,
               jnp.reshape(b1, (1, HIDDEN)),
               jnp.reshape(W2, (1, HIDDEN)),
               jnp.reshape(b2, (1, 1)))
    return jnp.squeeze(out, axis=-1)


# speedup vs baseline: 2.2840x; 1.2214x over previous
import functools

import jax
import jax.numpy as jnp
from jax import lax
from jax.experimental import pallas as pl
from jax.experimental.pallas import tpu as pltpu
from jax.experimental.pallas import tpu_sc as plsc

B = 4096
H = 20
E = 64
V = 100000
NTAB = 5
DENSE = 256
HIDDEN = 512

VP = 102400
PHALF = VP // 2
TBLK = 2048

NC, NS, L = 2, 16, 16
NW = NC * NS
BAGS_W = B // NW
CHUNK = 32
NCHUNK = BAGS_W // CHUNK
ROWS_CHUNK = CHUNK * H


def _sc_pool(idx, tab):
    mesh = plsc.VectorSubcoreMesh(core_axis_name="c", subcore_axis_name="s")

    @functools.partial(
        pl.kernel,
        out_type=jax.ShapeDtypeStruct((B, E), jnp.float32),
        mesh=mesh,
        scratch_types=[
            pltpu.VMEM((BAGS_W * H,), jnp.int32),
            pltpu.VMEM((2, ROWS_CHUNK, E), jnp.float32),
            pltpu.VMEM((BAGS_W, E), jnp.float32),
            pltpu.SemaphoreType.DMA,
            pltpu.SemaphoreType.DMA,
            pltpu.SemaphoreType.DMA,
        ],
        compiler_params=pltpu.CompilerParams(use_tc_tiling_on_sc=False),
    )
    def k(ihbm, thbm, out_hbm, idx_v, rows_v, pool_v, sem0, sem1, osem):
        wid = lax.axis_index("s") * NC + lax.axis_index("c")
        base_bag = wid * BAGS_W
        base_idx = base_bag * H
        sems = (sem0, sem1)
        pltpu.sync_copy(ihbm.at[pl.ds(base_idx, BAGS_W * H)], idx_v)

        def start(c, slot):
            return pltpu.async_copy(
                thbm.at[idx_v.at[pl.ds(c * ROWS_CHUNK, ROWS_CHUNK)]],
                rows_v.at[slot], sems[slot])

        handles = {0: start(0, 0)}
        for c in range(NCHUNK):
            slot = c % 2
            if c + 1 < NCHUNK:
                handles[c + 1] = start(c + 1, 1 - slot)
            handles.pop(c).wait()

            def body(bag, carry, _c=c, _slot=slot):
                r0 = bag * H
                for j in range(E // L):
                    acc = rows_v[_slot, r0, pl.ds(j * L, L)]
                    for q in range(1, H):
                        acc = acc + rows_v[_slot, r0 + q, pl.ds(j * L, L)]
                    pool_v[_c * CHUNK + bag, pl.ds(j * L, L)] = acc
                return carry

            lax.fori_loop(0, CHUNK, body, 0)
        pltpu.async_copy(pool_v, out_hbm.at[pl.ds(base_bag, BAGS_W)],
                         osem).wait()

    return k(idx, tab)


def _tc_repack(tT, eye):
    def body(x1_ref, x2_ref, i_ref, o_ref):
        dn = (((0,), (0,)), ((), ()))
        o_ref[:, 0:E] = jax.lax.dot_general(
            x1_ref[...], i_ref[...], dn, preferred_element_type=jnp.float32)
        o_ref[:, E:2 * E] = jax.lax.dot_general(
            x2_ref[...], i_ref[...], dn, preferred_element_type=jnp.float32)

    nblk = PHALF // TBLK
    last = -(-V // TBLK) - 1
    return pl.pallas_call(
        body,
        grid=(nblk,),
        in_specs=[
            pl.BlockSpec((E, TBLK), lambda i: (0, i)),
            pl.BlockSpec((E, TBLK), lambda i: (0, jnp.minimum(i + nblk, last))),
            pl.BlockSpec((E, E), lambda i: (0, 0)),
        ],
        out_specs=pl.BlockSpec((TBLK, 2 * E), lambda i: (i, 0)),
        out_shape=jax.ShapeDtypeStruct((PHALF, 2 * E), jnp.float32),
    )(tT, tT, eye)


def _mlp(pooled, dense, W1, b1r, W2r, b2r):
    BLK = 512
    FUSED = NTAB * E + DENSE

    def body(p0, p1, p2, p3, p4, d_ref, w1_ref, b1_ref, w2_ref, b2_ref, o_ref):
        h = jnp.dot(d_ref[...], w1_ref[NTAB * E:, :],
                    preferred_element_type=jnp.float32)
        for t, p_ref in enumerate((p0, p1, p2, p3, p4)):
            h = h + jnp.dot(p_ref[...], w1_ref[t * E:(t + 1) * E, :],
                            preferred_element_type=jnp.float32)
        h = h + b1_ref[...]
        h = h * jax.nn.sigmoid(h)
        o_ref[...] = jnp.sum(h * w2_ref[...], axis=1, keepdims=True) + b2_ref[...]

    return pl.pallas_call(
        body,
        grid=(B // BLK,),
        in_specs=[pl.BlockSpec((BLK, E), lambda i: (i, 0))] * NTAB + [
            pl.BlockSpec((BLK, DENSE), lambda i: (i, 0)),
            pl.BlockSpec((FUSED, HIDDEN), lambda i: (0, 0)),
            pl.BlockSpec((1, HIDDEN), lambda i: (0, 0)),
            pl.BlockSpec((1, HIDDEN), lambda i: (0, 0)),
            pl.BlockSpec((1, 1), lambda i: (0, 0)),
        ],
        out_specs=pl.BlockSpec((BLK, 1), lambda i: (i, 0)),
        out_shape=jax.ShapeDtypeStruct((B, 1), jnp.float32),
    )(*pooled, dense, W1, b1r, W2r, b2r)


def kernel(user_tokens, context_tokens, candidate_tokens,
           candidate_post_tokens, candidate_author_tokens, dense_features,
           table_user_tokens, table_context_tokens, table_candidate_tokens,
           table_candidate_post_tokens, table_candidate_author_tokens,
           W1, b1, W2, b2):
    idx = []
    for t in (user_tokens, context_tokens, candidate_tokens,
              candidate_post_tokens, candidate_author_tokens):
        t = t.astype(jnp.int32)
        t = jnp.where(t < PHALF, t * 2, t * 2 - (VP - 1))
        idx.append(jnp.reshape(t, (B * H,)))
    eye = jnp.eye(E, dtype=jnp.float32)
    pooled = []
    for i, tbl in enumerate((table_user_tokens, table_context_tokens,
                             table_candidate_tokens,
                             table_candidate_post_tokens,
                             table_candidate_author_tokens)):
        tab = jnp.reshape(_tc_repack(tbl.T, eye), (VP, E))
        pooled.append(_sc_pool(idx[i], tab))
    out = _mlp(pooled, dense_features, W1,
               jnp.reshape(b1, (1, HIDDEN)),
               jnp.reshape(W2, (1, HIDDEN)),
               jnp.reshape(b2, (1, 1)))
    return jnp.squeeze(out, axis=-1)


# speedup vs baseline: 2.3778x; 1.0411x over previous
import functools

import jax
import jax.numpy as jnp
from jax import lax
from jax.experimental import pallas as pl
from jax.experimental.pallas import tpu as pltpu
from jax.experimental.pallas import tpu_sc as plsc

B = 4096
H = 20
E = 64
V = 100000
NTAB = 5
DENSE = 256
HIDDEN = 512

VP = 102400
Q = VP // 4
QBLK = 1024

NC, NS, L = 2, 16, 16
NW = NC * NS
BAGS_W = B // NW
CHUNK = 32
NCHUNK = BAGS_W // CHUNK
ROWS_CHUNK = CHUNK * H


def _sc_pool(idx, tab):
    mesh = plsc.VectorSubcoreMesh(core_axis_name="c", subcore_axis_name="s")

    @functools.partial(
        pl.kernel,
        out_type=jax.ShapeDtypeStruct((B, E), jnp.float32),
        mesh=mesh,
        scratch_types=[
            pltpu.VMEM((BAGS_W * H,), jnp.int32),
            pltpu.VMEM((2, ROWS_CHUNK, E // 2), jnp.int32),
            pltpu.VMEM((BAGS_W, E), jnp.float32),
            pltpu.SemaphoreType.DMA,
            pltpu.SemaphoreType.DMA,
            pltpu.SemaphoreType.DMA,
        ],
        compiler_params=pltpu.CompilerParams(use_tc_tiling_on_sc=False,
                                             needs_layout_passes=False),
    )
    def k(ihbm, thbm, out_hbm, idx_v, rows_v, pool_v, sem0, sem1, osem):
        wid = lax.axis_index("s") * NC + lax.axis_index("c")
        base_bag = wid * BAGS_W
        base_idx = base_bag * H
        sems = (sem0, sem1)
        pltpu.sync_copy(ihbm.at[pl.ds(base_idx, BAGS_W * H)], idx_v)

        def start(c, slot):
            return pltpu.async_copy(
                thbm.at[idx_v.at[pl.ds(c * ROWS_CHUNK, ROWS_CHUNK)]],
                rows_v.at[slot], sems[slot])

        handles = {0: start(0, 0)}
        for c in range(NCHUNK):
            slot = c % 2
            if c + 1 < NCHUNK:
                handles[c + 1] = start(c + 1, 1 - slot)
            handles.pop(c).wait()

            def body(bag, carry, _c=c, _slot=slot):
                r0 = bag * H
                for j in range(E // (2 * L)):
                    words = rows_v[_slot, r0, pl.ds(j * L, L)]
                    acc_a, acc_b = plsc.unpack(
                        plsc.bitcast(words, jnp.bfloat16),
                        format=plsc.PackFormat.INTERLEAVED)
                    for q in range(1, H):
                        words = rows_v[_slot, r0 + q, pl.ds(j * L, L)]
                        a, b = plsc.unpack(
                            plsc.bitcast(words, jnp.bfloat16),
                            format=plsc.PackFormat.INTERLEAVED)
                        acc_a = acc_a + a
                        acc_b = acc_b + b
                    pool_v[_c * CHUNK + bag, pl.ds(2 * j * L, L)] = acc_a
                    pool_v[_c * CHUNK + bag, pl.ds((2 * j + 1) * L, L)] = acc_b
                return carry

            lax.fori_loop(0, CHUNK, body, 0)
        pltpu.async_copy(pool_v, out_hbm.at[pl.ds(base_bag, BAGS_W)],
                         osem).wait()

    return k(idx, tab)


def _tc_repack(tT, sel_even, sel_odd):
    def pack(x_ref, se, so):
        dn = (((0,), (0,)), ((), ()))
        xe = jax.lax.dot_general(x_ref[...], se, dn,
                                 preferred_element_type=jnp.float32)
        xo = jax.lax.dot_general(x_ref[...], so, dn,
                                 preferred_element_type=jnp.float32)
        we = jax.lax.bitcast_convert_type(
            xe.astype(jnp.bfloat16), jnp.uint16).astype(jnp.uint32)
        wo = jax.lax.bitcast_convert_type(
            xo.astype(jnp.bfloat16), jnp.uint16).astype(jnp.uint32)
        return jax.lax.bitcast_convert_type(we | (wo << 16), jnp.int32)

    def body(x1_ref, x2_ref, x3_ref, x4_ref, se_ref, so_ref, o_ref):
        se, so = se_ref[...], so_ref[...]
        for k, x_ref in enumerate((x1_ref, x2_ref, x3_ref, x4_ref)):
            o_ref[:, 32 * k:32 * (k + 1)] = pack(x_ref, se, so)

    nblk = Q // QBLK
    last = -(-V // QBLK) - 1

    def shifted(k):
        return lambda i: (0, jnp.minimum(i + k * (Q // QBLK), last))

    return pl.pallas_call(
        body,
        grid=(nblk,),
        in_specs=[
            pl.BlockSpec((E, QBLK), shifted(0)),
            pl.BlockSpec((E, QBLK), shifted(1)),
            pl.BlockSpec((E, QBLK), shifted(2)),
            pl.BlockSpec((E, QBLK), shifted(3)),
            pl.BlockSpec((E, E // 2), lambda i: (0, 0)),
            pl.BlockSpec((E, E // 2), lambda i: (0, 0)),
        ],
        out_specs=pl.BlockSpec((QBLK, 2 * E), lambda i: (i, 0)),
        out_shape=jax.ShapeDtypeStruct((Q, 2 * E), jnp.int32),
    )(tT, tT, tT, tT, sel_even, sel_odd)


def _mlp(pooled, dense, W1, b1r, W2r, b2r):
    BLK = 512
    FUSED = NTAB * E + DENSE

    def body(p0, p1, p2, p3, p4, d_ref, w1_ref, b1_ref, w2_ref, b2_ref, o_ref):
        h = jnp.dot(d_ref[...], w1_ref[NTAB * E:, :],
                    preferred_element_type=jnp.float32)
        for t, p_ref in enumerate((p0, p1, p2, p3, p4)):
            h = h + jnp.dot(p_ref[...], w1_ref[t * E:(t + 1) * E, :],
                            preferred_element_type=jnp.float32)
        h = h + b1_ref[...]
        h = h * jax.nn.sigmoid(h)
        o_ref[...] = jnp.sum(h * w2_ref[...], axis=1, keepdims=True) + b2_ref[...]

    return pl.pallas_call(
        body,
        grid=(B // BLK,),
        in_specs=[pl.BlockSpec((BLK, E), lambda i: (i, 0))] * NTAB + [
            pl.BlockSpec((BLK, DENSE), lambda i: (i, 0)),
            pl.BlockSpec((FUSED, HIDDEN), lambda i: (0, 0)),
            pl.BlockSpec((1, HIDDEN), lambda i: (0, 0)),
            pl.BlockSpec((1, HIDDEN), lambda i: (0, 0)),
            pl.BlockSpec((1, 1), lambda i: (0, 0)),
        ],
        out_specs=pl.BlockSpec((BLK, 1), lambda i: (i, 0)),
        out_shape=jax.ShapeDtypeStruct((B, 1), jnp.float32),
    )(*pooled, dense, W1, b1r, W2r, b2r)


def kernel(user_tokens, context_tokens, candidate_tokens,
           candidate_post_tokens, candidate_author_tokens, dense_features,
           table_user_tokens, table_context_tokens, table_candidate_tokens,
           table_candidate_post_tokens, table_candidate_author_tokens,
           W1, b1, W2, b2):
    idx = []
    for t in (user_tokens, context_tokens, candidate_tokens,
              candidate_post_tokens, candidate_author_tokens):
        t = t.astype(jnp.int32)
        t = (t % Q) * 4 + t // Q
        idx.append(jnp.reshape(t, (B * H,)))
    eye = jnp.eye(E, dtype=jnp.float32)
    sel_even, sel_odd = eye[:, 0::2], eye[:, 1::2]
    pooled = []
    for i, tbl in enumerate((table_user_tokens, table_context_tokens,
                             table_candidate_tokens,
                             table_candidate_post_tokens,
                             table_candidate_author_tokens)):
        tab = jnp.reshape(_tc_repack(tbl.T, sel_even, sel_odd), (VP, E // 2))
        pooled.append(_sc_pool(idx[i], tab))
    perm = (list(range(0, 32, 2)) + list(range(1, 32, 2))
            + list(range(32, 64, 2)) + list(range(33, 64, 2)))
    w1p = jnp.concatenate(
        [W1[t * E:(t + 1) * E][jnp.array(perm)] for t in range(NTAB)]
        + [W1[NTAB * E:]], axis=0)
    out = _mlp(pooled, dense_features, w1p,
               jnp.reshape(b1, (1, HIDDEN)),
               jnp.reshape(W2, (1, HIDDEN)),
               jnp.reshape(b2, (1, 1)))
    return jnp.squeeze(out, axis=-1)


# speedup vs baseline: 2.3902x; 1.0052x over previous
import functools

import jax
import jax.numpy as jnp
from jax import lax
from jax.experimental import pallas as pl
from jax.experimental.pallas import tpu as pltpu
from jax.experimental.pallas import tpu_sc as plsc

B = 4096
H = 20
E = 64
V = 100000
NTAB = 5
DENSE = 256
HIDDEN = 512

VP = 102400
Q = VP // 4
QBLK = 1024

NC, NS, L = 2, 16, 16
NW = NC * NS
BAGS_W = B // NW
CHUNK = 32
NCHUNK = BAGS_W // CHUNK
ROWS_CHUNK = CHUNK * H


def _sc_pool(idx, tab):
    mesh = plsc.VectorSubcoreMesh(core_axis_name="c", subcore_axis_name="s")

    @functools.partial(
        pl.kernel,
        out_type=jax.ShapeDtypeStruct((B, E), jnp.float32),
        mesh=mesh,
        scratch_types=[
            pltpu.VMEM((BAGS_W * H,), jnp.int32),
            pltpu.VMEM((2, ROWS_CHUNK, E // 2), jnp.int32),
            pltpu.VMEM((BAGS_W, E), jnp.float32),
            pltpu.SemaphoreType.DMA,
            pltpu.SemaphoreType.DMA,
            pltpu.SemaphoreType.DMA,
        ],
        compiler_params=pltpu.CompilerParams(use_tc_tiling_on_sc=False,
                                             needs_layout_passes=False),
    )
    def k(ihbm, thbm, out_hbm, idx_v, rows_v, pool_v, sem0, sem1, osem):
        wid = lax.axis_index("s") * NC + lax.axis_index("c")
        base_bag = wid * BAGS_W
        base_idx = base_bag * H
        sems = (sem0, sem1)
        pltpu.sync_copy(ihbm.at[pl.ds(base_idx, BAGS_W * H)], idx_v)

        def start(c, slot):
            return pltpu.async_copy(
                thbm.at[idx_v.at[pl.ds(c * ROWS_CHUNK, ROWS_CHUNK)]],
                rows_v.at[slot], sems[slot])

        handles = {0: start(0, 0)}
        for c in range(NCHUNK):
            slot = c % 2
            if c + 1 < NCHUNK:
                handles[c + 1] = start(c + 1, 1 - slot)
            handles.pop(c).wait()

            def body(bag, carry, _c=c, _slot=slot):
                r0 = bag * H
                for j in range(E // (2 * L)):
                    words = rows_v[_slot, r0, pl.ds(j * L, L)]
                    acc_a, acc_b = plsc.unpack(
                        plsc.bitcast(words, jnp.bfloat16),
                        format=plsc.PackFormat.INTERLEAVED)
                    for q in range(1, H):
                        words = rows_v[_slot, r0 + q, pl.ds(j * L, L)]
                        a, b = plsc.unpack(
                            plsc.bitcast(words, jnp.bfloat16),
                            format=plsc.PackFormat.INTERLEAVED)
                        acc_a = acc_a + a
                        acc_b = acc_b + b
                    pool_v[_c * CHUNK + bag, pl.ds(2 * j * L, L)] = acc_a
                    pool_v[_c * CHUNK + bag, pl.ds((2 * j + 1) * L, L)] = acc_b
                return carry

            lax.fori_loop(0, CHUNK, body, 0)
        pltpu.async_copy(pool_v, out_hbm.at[pl.ds(base_bag, BAGS_W)],
                         osem).wait()

    return k(idx, tab)


def _tc_repack(tT):
    def pack(x_ref):
        xt = jnp.swapaxes(x_ref[...], 0, 1)
        we = jax.lax.bitcast_convert_type(
            xt[:, 0:E // 2].astype(jnp.bfloat16), jnp.uint16).astype(jnp.uint32)
        wo = jax.lax.bitcast_convert_type(
            xt[:, E // 2:E].astype(jnp.bfloat16), jnp.uint16).astype(jnp.uint32)
        return jax.lax.bitcast_convert_type(we | (wo << 16), jnp.int32)

    def body(x1_ref, x2_ref, x3_ref, x4_ref, o_ref):
        for k, x_ref in enumerate((x1_ref, x2_ref, x3_ref, x4_ref)):
            o_ref[:, 32 * k:32 * (k + 1)] = pack(x_ref)

    nblk = Q // QBLK
    last = -(-V // QBLK) - 1

    def shifted(k):
        return lambda i: (0, jnp.minimum(i + k * (Q // QBLK), last))

    return pl.pallas_call(
        body,
        grid=(nblk,),
        in_specs=[
            pl.BlockSpec((E, QBLK), shifted(0)),
            pl.BlockSpec((E, QBLK), shifted(1)),
            pl.BlockSpec((E, QBLK), shifted(2)),
            pl.BlockSpec((E, QBLK), shifted(3)),
        ],
        out_specs=pl.BlockSpec((QBLK, 2 * E), lambda i: (i, 0)),
        out_shape=jax.ShapeDtypeStruct((Q, 2 * E), jnp.int32),
    )(tT, tT, tT, tT)


def _mlp(pooled, dense, W1, b1r, W2r, b2r):
    BLK = 512
    FUSED = NTAB * E + DENSE

    def body(p0, p1, p2, p3, p4, d_ref, w1_ref, b1_ref, w2_ref, b2_ref, o_ref):
        h = jnp.dot(d_ref[...], w1_ref[NTAB * E:, :],
                    preferred_element_type=jnp.float32)
        for t, p_ref in enumerate((p0, p1, p2, p3, p4)):
            h = h + jnp.dot(p_ref[...], w1_ref[t * E:(t + 1) * E, :],
                            preferred_element_type=jnp.float32)
        h = h + b1_ref[...]
        h = h * jax.nn.sigmoid(h)
        o_ref[...] = jnp.sum(h * w2_ref[...], axis=1, keepdims=True) + b2_ref[...]

    return pl.pallas_call(
        body,
        grid=(B // BLK,),
        in_specs=[pl.BlockSpec((BLK, E), lambda i: (i, 0))] * NTAB + [
            pl.BlockSpec((BLK, DENSE), lambda i: (i, 0)),
            pl.BlockSpec((FUSED, HIDDEN), lambda i: (0, 0)),
            pl.BlockSpec((1, HIDDEN), lambda i: (0, 0)),
            pl.BlockSpec((1, HIDDEN), lambda i: (0, 0)),
            pl.BlockSpec((1, 1), lambda i: (0, 0)),
        ],
        out_specs=pl.BlockSpec((BLK, 1), lambda i: (i, 0)),
        out_shape=jax.ShapeDtypeStruct((B, 1), jnp.float32),
    )(*pooled, dense, W1, b1r, W2r, b2r)


def kernel(user_tokens, context_tokens, candidate_tokens,
           candidate_post_tokens, candidate_author_tokens, dense_features,
           table_user_tokens, table_context_tokens, table_candidate_tokens,
           table_candidate_post_tokens, table_candidate_author_tokens,
           W1, b1, W2, b2):
    idx = []
    for t in (user_tokens, context_tokens, candidate_tokens,
              candidate_post_tokens, candidate_author_tokens):
        t = t.astype(jnp.int32)
        t = (t % Q) * 4 + t // Q
        idx.append(jnp.reshape(t, (B * H,)))
    pooled = []
    for i, tbl in enumerate((table_user_tokens, table_context_tokens,
                             table_candidate_tokens,
                             table_candidate_post_tokens,
                             table_candidate_author_tokens)):
        tab = jnp.reshape(_tc_repack(tbl.T), (VP, E // 2))
        pooled.append(_sc_pool(idx[i], tab))
    perm = (list(range(0, 16)) + list(range(32, 48))
            + list(range(16, 32)) + list(range(48, 64)))
    w1p = jnp.concatenate(
        [W1[t * E:(t + 1) * E][jnp.array(perm)] for t in range(NTAB)]
        + [W1[NTAB * E:]], axis=0)
    out = _mlp(pooled, dense_features, w1p,
               jnp.reshape(b1, (1, HIDDEN)),
               jnp.reshape(W2, (1, HIDDEN)),
               jnp.reshape(b2, (1, 1)))
    return jnp.squeeze(out, axis=-1)


# speedup vs baseline: 2.5037x; 1.0475x over previous
import functools

import jax
import jax.numpy as jnp
from jax import lax
from jax.experimental import pallas as pl
from jax.experimental.pallas import tpu as pltpu
from jax.experimental.pallas import tpu_sc as plsc

B = 4096
H = 20
E = 64
V = 100000
NTAB = 5
DENSE = 256
HIDDEN = 512

VP = 102400
Q = VP // 4
QBLK = 1280

NC, NS, L = 2, 16, 16
NW = NC * NS
BAGS_W = B // NW
CHUNK = 32
NCHUNK = BAGS_W // CHUNK
ROWS_CHUNK = CHUNK * H


def _sc_pool(idx, tab):
    mesh = plsc.VectorSubcoreMesh(core_axis_name="c", subcore_axis_name="s")

    @functools.partial(
        pl.kernel,
        out_type=jax.ShapeDtypeStruct((B, E), jnp.float32),
        mesh=mesh,
        scratch_types=[
            pltpu.VMEM((BAGS_W * H,), jnp.int32),
            pltpu.VMEM((2, ROWS_CHUNK, E // 2), jnp.int32),
            pltpu.VMEM((BAGS_W, E), jnp.float32),
            pltpu.SemaphoreType.DMA,
            pltpu.SemaphoreType.DMA,
            pltpu.SemaphoreType.DMA,
        ],
        compiler_params=pltpu.CompilerParams(use_tc_tiling_on_sc=False,
                                             needs_layout_passes=False),
    )
    def k(ihbm, thbm, out_hbm, idx_v, rows_v, pool_v, sem0, sem1, osem):
        wid = lax.axis_index("s") * NC + lax.axis_index("c")
        base_bag = wid * BAGS_W
        base_idx = base_bag * H
        sems = (sem0, sem1)
        pltpu.sync_copy(ihbm.at[pl.ds(base_idx, BAGS_W * H)], idx_v)

        def start(c, slot):
            return pltpu.async_copy(
                thbm.at[idx_v.at[pl.ds(c * ROWS_CHUNK, ROWS_CHUNK)]],
                rows_v.at[slot], sems[slot])

        handles = {0: start(0, 0)}
        for c in range(NCHUNK):
            slot = c % 2
            if c + 1 < NCHUNK:
                handles[c + 1] = start(c + 1, 1 - slot)
            handles.pop(c).wait()

            def body(bag, carry, _c=c, _slot=slot):
                r0 = bag * H
                for j in range(E // (2 * L)):
                    words = rows_v[_slot, r0, pl.ds(j * L, L)]
                    acc_a, acc_b = plsc.unpack(
                        plsc.bitcast(words, jnp.bfloat16),
                        format=plsc.PackFormat.INTERLEAVED)
                    for q in range(1, H):
                        words = rows_v[_slot, r0 + q, pl.ds(j * L, L)]
                        a, b = plsc.unpack(
                            plsc.bitcast(words, jnp.bfloat16),
                            format=plsc.PackFormat.INTERLEAVED)
                        acc_a = acc_a + a
                        acc_b = acc_b + b
                    pool_v[_c * CHUNK + bag, pl.ds(2 * j * L, L)] = acc_a
                    pool_v[_c * CHUNK + bag, pl.ds((2 * j + 1) * L, L)] = acc_b
                return carry

            lax.fori_loop(0, CHUNK, body, 0)
        pltpu.async_copy(pool_v, out_hbm.at[pl.ds(base_bag, BAGS_W)],
                         osem).wait()

    return k(idx, tab)


def _tc_repack(tT):
    def pack(x_ref):
        xt = jnp.swapaxes(x_ref[...], 0, 1)
        we = jax.lax.bitcast_convert_type(
            xt[:, 0:E // 2].astype(jnp.bfloat16), jnp.uint16).astype(jnp.uint32)
        wo = jax.lax.bitcast_convert_type(
            xt[:, E // 2:E].astype(jnp.bfloat16), jnp.uint16).astype(jnp.uint32)
        return jax.lax.bitcast_convert_type(we | (wo << 16), jnp.int32)

    def body(x1_ref, x2_ref, x3_ref, x4_ref, o_ref):
        for k, x_ref in enumerate((x1_ref, x2_ref, x3_ref, x4_ref)):
            o_ref[:, 32 * k:32 * (k + 1)] = pack(x_ref)

    nblk = Q // QBLK
    last = -(-V // QBLK) - 1

    def shifted(k):
        return lambda i: (0, jnp.minimum(i + k * (Q // QBLK), last))

    return pl.pallas_call(
        body,
        grid=(nblk,),
        in_specs=[
            pl.BlockSpec((E, QBLK), shifted(0)),
            pl.BlockSpec((E, QBLK), shifted(1)),
            pl.BlockSpec((E, QBLK), shifted(2)),
            pl.BlockSpec((E, QBLK), shifted(3)),
        ],
        out_specs=pl.BlockSpec((QBLK, 2 * E), lambda i: (i, 0)),
        out_shape=jax.ShapeDtypeStruct((Q, 2 * E), jnp.int32),
    )(tT, tT, tT, tT)


def _mlp(pooled, dense, W1, b1r, W2r, b2r):
    BLK = 512
    FUSED = NTAB * E + DENSE

    def body(p0, p1, p2, p3, p4, d_ref, w1_ref, b1_ref, w2_ref, b2_ref, o_ref):
        h = jnp.dot(d_ref[...], w1_ref[NTAB * E:, :],
                    preferred_element_type=jnp.float32)
        for t, p_ref in enumerate((p0, p1, p2, p3, p4)):
            h = h + jnp.dot(p_ref[...], w1_ref[t * E:(t + 1) * E, :],
                            preferred_element_type=jnp.float32)
        h = h + b1_ref[...]
        h = h * jax.nn.sigmoid(h)
        o_ref[...] = jnp.sum(h * w2_ref[...], axis=1, keepdims=True) + b2_ref[...]

    return pl.pallas_call(
        body,
        grid=(B // BLK,),
        in_specs=[pl.BlockSpec((BLK, E), lambda i: (i, 0))] * NTAB + [
            pl.BlockSpec((BLK, DENSE), lambda i: (i, 0)),
            pl.BlockSpec((FUSED, HIDDEN), lambda i: (0, 0)),
            pl.BlockSpec((1, HIDDEN), lambda i: (0, 0)),
            pl.BlockSpec((1, HIDDEN), lambda i: (0, 0)),
            pl.BlockSpec((1, 1), lambda i: (0, 0)),
        ],
        out_specs=pl.BlockSpec((BLK, 1), lambda i: (i, 0)),
        out_shape=jax.ShapeDtypeStruct((B, 1), jnp.float32),
    )(*pooled, dense, W1, b1r, W2r, b2r)


def kernel(user_tokens, context_tokens, candidate_tokens,
           candidate_post_tokens, candidate_author_tokens, dense_features,
           table_user_tokens, table_context_tokens, table_candidate_tokens,
           table_candidate_post_tokens, table_candidate_author_tokens,
           W1, b1, W2, b2):
    idx = []
    for t in (user_tokens, context_tokens, candidate_tokens,
              candidate_post_tokens, candidate_author_tokens):
        t = t.astype(jnp.int32)
        t = (t % Q) * 4 + t // Q
        idx.append(jnp.reshape(t, (B * H,)))
    pooled = []
    for i, tbl in enumerate((table_user_tokens, table_context_tokens,
                             table_candidate_tokens,
                             table_candidate_post_tokens,
                             table_candidate_author_tokens)):
        tab = jnp.reshape(_tc_repack(tbl.T), (VP, E // 2))
        pooled.append(_sc_pool(idx[i], tab))
    perm = (list(range(0, 16)) + list(range(32, 48))
            + list(range(16, 32)) + list(range(48, 64)))
    w1p = jnp.concatenate(
        [W1[t * E:(t + 1) * E][jnp.array(perm)] for t in range(NTAB)]
        + [W1[NTAB * E:]], axis=0)
    out = _mlp(pooled, dense_features, w1p,
               jnp.reshape(b1, (1, HIDDEN)),
               jnp.reshape(W2, (1, HIDDEN)),
               jnp.reshape(b2, (1, 1)))
    return jnp.squeeze(out, axis=-1)


# speedup vs baseline: 2.6793x; 1.0701x over previous
import functools

import jax
import jax.numpy as jnp
from jax import lax
from jax.experimental import pallas as pl
from jax.experimental.pallas import tpu as pltpu
from jax.experimental.pallas import tpu_sc as plsc

B = 4096
H = 20
E = 64
V = 100000
NTAB = 5
DENSE = 256
HIDDEN = 512

VP = 102400
Q = VP // 4
QBLK = 2560

NC, NS, L = 2, 16, 16
NW = NC * NS
BAGS_W = B // NW
CHUNK = 32
NCHUNK = BAGS_W // CHUNK
ROWS_CHUNK = CHUNK * H


def _sc_pool(idx, tab):
    mesh = plsc.VectorSubcoreMesh(core_axis_name="c", subcore_axis_name="s")

    @functools.partial(
        pl.kernel,
        out_type=jax.ShapeDtypeStruct((B, E), jnp.float32),
        mesh=mesh,
        scratch_types=[
            pltpu.VMEM((BAGS_W * H,), jnp.int32),
            pltpu.VMEM((2, ROWS_CHUNK, E // 2), jnp.int32),
            pltpu.VMEM((BAGS_W, E), jnp.float32),
            pltpu.SemaphoreType.DMA,
            pltpu.SemaphoreType.DMA,
            pltpu.SemaphoreType.DMA,
        ],
        compiler_params=pltpu.CompilerParams(use_tc_tiling_on_sc=False,
                                             needs_layout_passes=False),
    )
    def k(ihbm, thbm, out_hbm, idx_v, rows_v, pool_v, sem0, sem1, osem):
        wid = lax.axis_index("s") * NC + lax.axis_index("c")
        base_bag = wid * BAGS_W
        base_idx = base_bag * H
        sems = (sem0, sem1)
        pltpu.sync_copy(ihbm.at[pl.ds(base_idx, BAGS_W * H)], idx_v)

        def start(c, slot):
            return pltpu.async_copy(
                thbm.at[idx_v.at[pl.ds(c * ROWS_CHUNK, ROWS_CHUNK)]],
                rows_v.at[slot], sems[slot])

        handles = {0: start(0, 0)}
        for c in range(NCHUNK):
            slot = c % 2
            if c + 1 < NCHUNK:
                handles[c + 1] = start(c + 1, 1 - slot)
            handles.pop(c).wait()

            def body(bag, carry, _c=c, _slot=slot):
                r0 = bag * H
                for j in range(E // (2 * L)):
                    words = rows_v[_slot, r0, pl.ds(j * L, L)]
                    acc_a, acc_b = plsc.unpack(
                        plsc.bitcast(words, jnp.bfloat16),
                        format=plsc.PackFormat.INTERLEAVED)
                    for q in range(1, H):
                        words = rows_v[_slot, r0 + q, pl.ds(j * L, L)]
                        a, b = plsc.unpack(
                            plsc.bitcast(words, jnp.bfloat16),
                            format=plsc.PackFormat.INTERLEAVED)
                        acc_a = acc_a + a
                        acc_b = acc_b + b
                    pool_v[_c * CHUNK + bag, pl.ds(2 * j * L, L)] = acc_a
                    pool_v[_c * CHUNK + bag, pl.ds((2 * j + 1) * L, L)] = acc_b
                return carry

            lax.fori_loop(0, CHUNK, body, 0)
        pltpu.async_copy(pool_v, out_hbm.at[pl.ds(base_bag, BAGS_W)],
                         osem).wait()

    return k(idx, tab)


def _tc_repack(tT):
    def pack(x_ref):
        xt = jnp.swapaxes(x_ref[...], 0, 1)
        we = jax.lax.bitcast_convert_type(
            xt[:, 0:E // 2].astype(jnp.bfloat16), jnp.uint16).astype(jnp.uint32)
        wo = jax.lax.bitcast_convert_type(
            xt[:, E // 2:E].astype(jnp.bfloat16), jnp.uint16).astype(jnp.uint32)
        return jax.lax.bitcast_convert_type(we | (wo << 16), jnp.int32)

    def body(x1_ref, x2_ref, x3_ref, x4_ref, o_ref):
        for k, x_ref in enumerate((x1_ref, x2_ref, x3_ref, x4_ref)):
            o_ref[:, 32 * k:32 * (k + 1)] = pack(x_ref)

    nblk = Q // QBLK
    last = -(-V // QBLK) - 1

    def shifted(k):
        return lambda i: (0, jnp.minimum(i + k * (Q // QBLK), last))

    return pl.pallas_call(
        body,
        grid=(nblk,),
        in_specs=[
            pl.BlockSpec((E, QBLK), shifted(0)),
            pl.BlockSpec((E, QBLK), shifted(1)),
            pl.BlockSpec((E, QBLK), shifted(2)),
            pl.BlockSpec((E, QBLK), shifted(3)),
        ],
        out_specs=pl.BlockSpec((QBLK, 2 * E), lambda i: (i, 0)),
        out_shape=jax.ShapeDtypeStruct((Q, 2 * E), jnp.int32),
    )(tT, tT, tT, tT)


def _mlp(pooled, dense, W1, b1r, W2r, b2r):
    BLK = 512
    FUSED = NTAB * E + DENSE

    def body(p0, p1, p2, p3, p4, d_ref, w1_ref, b1_ref, w2_ref, b2_ref, o_ref):
        h = jnp.dot(d_ref[...], w1_ref[NTAB * E:, :],
                    preferred_element_type=jnp.float32)
        for t, p_ref in enumerate((p0, p1, p2, p3, p4)):
            h = h + jnp.dot(p_ref[...], w1_ref[t * E:(t + 1) * E, :],
                            preferred_element_type=jnp.float32)
        h = h + b1_ref[...]
        h = h * jax.nn.sigmoid(h)
        o_ref[...] = jnp.sum(h * w2_ref[...], axis=1, keepdims=True) + b2_ref[...]

    return pl.pallas_call(
        body,
        grid=(B // BLK,),
        in_specs=[pl.BlockSpec((BLK, E), lambda i: (i, 0))] * NTAB + [
            pl.BlockSpec((BLK, DENSE), lambda i: (i, 0)),
            pl.BlockSpec((FUSED, HIDDEN), lambda i: (0, 0)),
            pl.BlockSpec((1, HIDDEN), lambda i: (0, 0)),
            pl.BlockSpec((1, HIDDEN), lambda i: (0, 0)),
            pl.BlockSpec((1, 1), lambda i: (0, 0)),
        ],
        out_specs=pl.BlockSpec((BLK, 1), lambda i: (i, 0)),
        out_shape=jax.ShapeDtypeStruct((B, 1), jnp.float32),
    )(*pooled, dense, W1, b1r, W2r, b2r)


def kernel(user_tokens, context_tokens, candidate_tokens,
           candidate_post_tokens, candidate_author_tokens, dense_features,
           table_user_tokens, table_context_tokens, table_candidate_tokens,
           table_candidate_post_tokens, table_candidate_author_tokens,
           W1, b1, W2, b2):
    idx = []
    for t in (user_tokens, context_tokens, candidate_tokens,
              candidate_post_tokens, candidate_author_tokens):
        t = t.astype(jnp.int32)
        t = (t % Q) * 4 + t // Q
        idx.append(jnp.reshape(t, (B * H,)))
    pooled = []
    for i, tbl in enumerate((table_user_tokens, table_context_tokens,
                             table_candidate_tokens,
                             table_candidate_post_tokens,
                             table_candidate_author_tokens)):
        tab = jnp.reshape(_tc_repack(tbl.T), (VP, E // 2))
        pooled.append(_sc_pool(idx[i], tab))
    perm = (list(range(0, 16)) + list(range(32, 48))
            + list(range(16, 32)) + list(range(48, 64)))
    w1p = jnp.concatenate(
        [W1[t * E:(t + 1) * E][jnp.array(perm)] for t in range(NTAB)]
        + [W1[NTAB * E:]], axis=0)
    out = _mlp(pooled, dense_features, w1p,
               jnp.reshape(b1, (1, HIDDEN)),
               jnp.reshape(W2, (1, HIDDEN)),
               jnp.reshape(b2, (1, 1)))
    return jnp.squeeze(out, axis=-1)
